# baseline (speedup 1.0000x reference)
import functools

import jax
import jax.numpy as jnp
from jax import lax
from jax.experimental import pallas as pl
from jax.experimental.pallas import tpu as pltpu
from jax.experimental.pallas import tpu_sc as plsc

_P = 262144
_D = 64
_NB = 32
_NW = 32
_PW = _P // _NW
_CH = 128
_NCH = _PW // _CH
_NPV = _CH // 16
_NBUF = 4

_DNUMS = lax.GatherDimensionNumbers(
    offset_dims=(), collapsed_slice_dims=(0,), start_index_map=(0,)
)


def _take16(vec, idx):
    return lax.gather(
        vec, idx[:, None], _DNUMS, slice_sizes=(1,),
        mode=lax.GatherScatterMode.PROMISE_IN_BOUNDS,
    )


def _sc_lookup(scale_vec, shift_vec, base_cantor, basis_t):
    mesh = plsc.VectorSubcoreMesh(core_axis_name="c", subcore_axis_name="s")

    @functools.partial(
        pl.kernel,
        out_type=jax.ShapeDtypeStruct((_D, _P), jnp.float32),
        mesh=mesh,
        scratch_types=[
            pltpu.VMEM((16,), jnp.float32),
            pltpu.VMEM((16,), jnp.float32),
            pltpu.VMEM((_PW,), jnp.float32),
            pltpu.VMEM((_D, _NB), jnp.float32),
            [pltpu.VMEM((_D, _CH), jnp.float32) for _ in range(_NBUF)],
            [pltpu.SemaphoreType.DMA for _ in range(_NBUF)],
        ],
    )
    def body(scale_hbm, shift_hbm, cantor_hbm, tabt_hbm, outt_hbm,
             scale_v, shift_v, cant_v, tabt_v, bufs, osem):
        wid = lax.axis_index("s") * 2 + lax.axis_index("c")
        base = wid * _PW
        pltpu.sync_copy(scale_hbm, scale_v)
        pltpu.sync_copy(shift_hbm, shift_v)
        pltpu.sync_copy(tabt_hbm, tabt_v)
        pltpu.sync_copy(cantor_hbm.at[pl.ds(base, _PW)], cant_v)
        s = scale_v[...]
        t = shift_v[...]

        def outer(jj, carry):
            for b in range(_NBUF):
                j = jj * _NBUF + b
                buf = bufs[b]

                @pl.when(jj > 0)
                def _wait():
                    pltpu.make_async_copy(
                        buf, outt_hbm.at[:, pl.ds(base, _CH)], osem[b]
                    ).wait()

                ilos = []
                msks = []
                for pv in range(_NPV):
                    c = cant_v[pl.ds(j * _CH + pv * 16, 16)]
                    iv = (c * s + t).astype(jnp.int32) & (_NB - 1)
                    ilos.append(iv & 15)
                    msks.append(iv < 16)

                def dloop(d, carry2):
                    lo = tabt_v[d, pl.ds(0, 16)]
                    hi = tabt_v[d, pl.ds(16, 16)]
                    for pv in range(_NPV):
                        g = jnp.where(
                            msks[pv], _take16(lo, ilos[pv]), _take16(hi, ilos[pv])
                        )
                        buf[d, pl.ds(pv * 16, 16)] = g
                    return carry2

                lax.fori_loop(0, _D, dloop, 0)
                pltpu.async_copy(
                    buf, outt_hbm.at[:, pl.ds(base + j * _CH, _CH)], osem[b]
                )
            return carry

        lax.fori_loop(0, _NCH // _NBUF, outer, 0)
        for b in range(_NBUF):
            pltpu.make_async_copy(
                bufs[b], outt_hbm.at[:, pl.ds(base, _CH)], osem[b]
            ).wait()

    return body(scale_vec, shift_vec, base_cantor, basis_t)


def kernel(fingerprint, basis, W, b, base_cantor):
    params = W @ fingerprint + b
    scale = jax.nn.sigmoid(params[0]) * 2.0 + 0.5
    shift = jnp.sum(jnp.tanh(params[1:2]) * 512.0)
    scale_vec = jnp.full((16,), scale, jnp.float32)
    shift_vec = jnp.full((16,), shift, jnp.float32)
    out_t = _sc_lookup(scale_vec, shift_vec, base_cantor, basis.T)
    return out_t.T


# speedup vs baseline: 1.0248x; 1.0248x over previous
import functools

import jax
import jax.numpy as jnp
from jax import lax
from jax.experimental import pallas as pl
from jax.experimental.pallas import tpu as pltpu
from jax.experimental.pallas import tpu_sc as plsc

_P = 262144
_D = 64
_NB = 32
_NW = 32
_PW = _P // _NW
_CH = 128
_NCH = _PW // _CH
_NPV = _CH // 16
_NBUF = 4

_DNUMS = lax.GatherDimensionNumbers(
    offset_dims=(), collapsed_slice_dims=(0,), start_index_map=(0,)
)


def _take16(vec, idx):
    return lax.gather(
        vec, idx[:, None], _DNUMS, slice_sizes=(1,),
        mode=lax.GatherScatterMode.PROMISE_IN_BOUNDS,
    )


def _sc_lookup(scale_vec, shift_vec, base_cantor, basis_t):
    mesh = plsc.VectorSubcoreMesh(core_axis_name="c", subcore_axis_name="s")

    @functools.partial(
        pl.kernel,
        out_type=jax.ShapeDtypeStruct((_D, _P), jnp.float32),
        mesh=mesh,
        scratch_types=[
            pltpu.VMEM((16,), jnp.float32),
            pltpu.VMEM((16,), jnp.float32),
            pltpu.VMEM((_PW,), jnp.float32),
            pltpu.VMEM((_D, _NB), jnp.float32),
            [pltpu.VMEM((_D, _CH), jnp.float32) for _ in range(_NBUF)],
            [pltpu.SemaphoreType.DMA for _ in range(_NBUF)],
        ],
    )
    def body(scale_hbm, shift_hbm, cantor_hbm, tabt_hbm, outt_hbm,
             scale_v, shift_v, cant_v, tabt_v, bufs, osem):
        wid = lax.axis_index("s") * 2 + lax.axis_index("c")
        base = wid * _PW
        pltpu.sync_copy(scale_hbm, scale_v)
        pltpu.sync_copy(shift_hbm, shift_v)
        pltpu.sync_copy(tabt_hbm, tabt_v)
        pltpu.sync_copy(cantor_hbm.at[pl.ds(base, _PW)], cant_v)
        s = scale_v[...]
        t = shift_v[...]

        def outer(jj, carry):
            for b in range(_NBUF):
                j = jj * _NBUF + b
                buf = bufs[b]

                @pl.when(jj > 0)
                def _wait():
                    pltpu.make_async_copy(
                        buf, outt_hbm.at[:, pl.ds(base, _CH)], osem[b]
                    ).wait()

                ilos = []
                msks = []
                for pv in range(_NPV):
                    c = cant_v[pl.ds(j * _CH + pv * 16, 16)]
                    iv = (c * s + t).astype(jnp.int32) & (_NB - 1)
                    ilos.append(iv & 15)
                    msks.append(iv < 16)

                def dloop(d2, carry2):
                    for dd in range(2):
                        d = d2 * 2 + dd
                        lo = tabt_v[d, pl.ds(0, 16)]
                        hi = tabt_v[d, pl.ds(16, 16)]
                        for pv in range(_NPV):
                            g = jnp.where(
                                msks[pv], _take16(lo, ilos[pv]), _take16(hi, ilos[pv])
                            )
                            buf[d, pl.ds(pv * 16, 16)] = g
                    return carry2

                lax.fori_loop(0, _D // 2, dloop, 0)
                pltpu.async_copy(
                    buf, outt_hbm.at[:, pl.ds(base + j * _CH, _CH)], osem[b]
                )
            return carry

        lax.fori_loop(0, _NCH // _NBUF, outer, 0)
        for b in range(_NBUF):
            pltpu.make_async_copy(
                bufs[b], outt_hbm.at[:, pl.ds(base, _CH)], osem[b]
            ).wait()

    return body(scale_vec, shift_vec, base_cantor, basis_t)


def kernel(fingerprint, basis, W, b, base_cantor):
    params = W @ fingerprint + b
    scale = jax.nn.sigmoid(params[0]) * 2.0 + 0.5
    shift = jnp.sum(jnp.tanh(params[1:2]) * 512.0)
    scale_vec = jnp.full((16,), scale, jnp.float32)
    shift_vec = jnp.full((16,), shift, jnp.float32)
    out_t = _sc_lookup(scale_vec, shift_vec, base_cantor, basis.T)
    return out_t.T


# speedup vs baseline: 1.0313x; 1.0063x over previous
import functools

import jax
import jax.numpy as jnp
from jax import lax
from jax.experimental import pallas as pl
from jax.experimental.pallas import tpu as pltpu
from jax.experimental.pallas import tpu_sc as plsc

_P = 262144
_D = 64
_NB = 32
_NW = 32
_PW = _P // _NW
_CH = 128
_NCH = _PW // _CH
_NPV = _CH // 16
_NBUF = 4

_DNUMS = lax.GatherDimensionNumbers(
    offset_dims=(), collapsed_slice_dims=(0,), start_index_map=(0,)
)


def _take16(vec, idx):
    return lax.gather(
        vec, idx[:, None], _DNUMS, slice_sizes=(1,),
        mode=lax.GatherScatterMode.PROMISE_IN_BOUNDS,
    )


def _sc_lookup(scale_vec, shift_vec, base_cantor, basis_t):
    mesh = plsc.VectorSubcoreMesh(core_axis_name="c", subcore_axis_name="s")

    @functools.partial(
        pl.kernel,
        out_type=jax.ShapeDtypeStruct((_D, _P), jnp.float32),
        mesh=mesh,
        scratch_types=[
            pltpu.VMEM((16,), jnp.float32),
            pltpu.VMEM((16,), jnp.float32),
            pltpu.VMEM((_PW,), jnp.float32),
            pltpu.VMEM((_D, _NB), jnp.float32),
            [pltpu.VMEM((_D, _CH), jnp.float32) for _ in range(_NBUF)],
            [pltpu.SemaphoreType.DMA for _ in range(_NBUF)],
        ],
    )
    def body(scale_hbm, shift_hbm, cantor_hbm, tabt_hbm, outt_hbm,
             scale_v, shift_v, cant_v, tabt_v, bufs, osem):
        wid = lax.axis_index("s") * 2 + lax.axis_index("c")
        base = wid * _PW
        pltpu.sync_copy(scale_hbm, scale_v)
        pltpu.sync_copy(shift_hbm, shift_v)
        pltpu.sync_copy(tabt_hbm, tabt_v)
        pltpu.sync_copy(cantor_hbm.at[pl.ds(base, _PW)], cant_v)
        s = scale_v[...]
        t = shift_v[...]

        def outer(jj, carry):
            for b in range(_NBUF):
                j = jj * _NBUF + b
                buf = bufs[b]

                @pl.when(jj > 0)
                def _wait():
                    pltpu.make_async_copy(
                        buf, outt_hbm.at[:, pl.ds(base, _CH)], osem[b]
                    ).wait()

                ilos = []
                msks = []
                for pv in range(_NPV):
                    c = cant_v[pl.ds(j * _CH + pv * 16, 16)]
                    iv = (c * s + t).astype(jnp.int32) & (_NB - 1)
                    ilos.append(iv & 15)
                    msks.append(iv < 16)

                def dloop(d2, carry2):
                    for dd in range(4):
                        d = d2 * 4 + dd
                        lo = tabt_v[d, pl.ds(0, 16)]
                        hi = tabt_v[d, pl.ds(16, 16)]
                        for pv in range(_NPV):
                            g = jnp.where(
                                msks[pv], _take16(lo, ilos[pv]), _take16(hi, ilos[pv])
                            )
                            buf[d, pl.ds(pv * 16, 16)] = g
                    return carry2

                lax.fori_loop(0, _D // 4, dloop, 0)
                pltpu.async_copy(
                    buf, outt_hbm.at[:, pl.ds(base + j * _CH, _CH)], osem[b]
                )
            return carry

        lax.fori_loop(0, _NCH // _NBUF, outer, 0)
        for b in range(_NBUF):
            pltpu.make_async_copy(
                bufs[b], outt_hbm.at[:, pl.ds(base, _CH)], osem[b]
            ).wait()

    return body(scale_vec, shift_vec, base_cantor, basis_t)


def kernel(fingerprint, basis, W, b, base_cantor):
    params = W @ fingerprint + b
    scale = jax.nn.sigmoid(params[0]) * 2.0 + 0.5
    shift = jnp.sum(jnp.tanh(params[1:2]) * 512.0)
    scale_vec = jnp.full((16,), scale, jnp.float32)
    shift_vec = jnp.full((16,), shift, jnp.float32)
    out_t = _sc_lookup(scale_vec, shift_vec, base_cantor, basis.T)
    return out_t.T
